# baseline (speedup 1.0000x reference)
import functools
import jax
import jax.numpy as jnp
from jax import lax
from jax.experimental import pallas as pl
from jax.experimental.pallas import tpu as pltpu
from jax.experimental.pallas import tpu_sc as plsc

NEG = jnp.finfo(jnp.float32).min

_N = 8192
_NW = 32
_SC_ROWS = 2048
_RPW = _SC_ROWS // _NW
_RC = 4
_NCH = _RPW // _RC
_LANES = 16
_KV = _N // _LANES
_KU = 4

_BR = 256
_BR2 = (_N - _SC_ROWS) // (_N // _BR)


def _sc_sum_body(adj_hbm, f1_hbm, out_hbm, f1_v, buf0, buf1, out_v, acc_buf,
                 sem0, sem1):
    wid = lax.axis_index("s") * 2 + lax.axis_index("c")
    base = wid * _RPW
    pltpu.sync_copy(f1_hbm, f1_v)
    bufs = (buf0, buf1)
    sems = (sem0, sem1)
    lanes = lax.iota(jnp.int32, _LANES)

    pltpu.async_copy(adj_hbm.at[pl.ds(base, _RC)], buf0, sem0)

    def group_body(g, _):
        for cc in range(_LANES // _RC):
            c = g * (_LANES // _RC) + cc
            p = cc % 2
            buf = bufs[p]
            row0 = base + c * _RC
            pltpu.make_async_copy(adj_hbm.at[pl.ds(row0, _RC)], buf, sems[p]).wait()

            @pl.when(c + 1 < _NCH)
            def _prefetch():
                pltpu.async_copy(
                    adj_hbm.at[pl.ds(row0 + _RC, _RC)], bufs[1 - p], sems[1 - p])

            def kbody(k, accs):
                accs = list(accs)
                for u in range(_KU):
                    off = (k * _KU + u) * _LANES
                    f = f1_v[pl.ds(off, _LANES)]
                    for r in range(_RC):
                        a = buf[r, pl.ds(off, _LANES)].astype(jnp.float32)
                        accs[r] = accs[r] + a * f
                return tuple(accs)

            accs = lax.fori_loop(
                0, _KV // _KU, kbody, tuple(jnp.zeros((_LANES,), jnp.float32)
                                            for _ in range(_RC)))
            for r in range(_RC):
                acc_buf[cc * _RC + r, :] = accs[r]
        res = jnp.zeros((_LANES,), jnp.float32)
        for t in range(_LANES):
            col = jnp.full((_LANES,), t, jnp.int32)
            res = res + plsc.load_gather(acc_buf, [lanes, col])
        out_v[pl.ds(g * _LANES, _LANES)] = res
        return 0

    lax.fori_loop(0, _RPW // _LANES, group_body, 0)
    pltpu.sync_copy(out_v, out_hbm.at[pl.ds(base, _RPW)])


def _sc_scal2_top(adj_1to2, f1_row):
    mesh = plsc.VectorSubcoreMesh(core_axis_name="c", subcore_axis_name="s")
    return pl.kernel(
        _sc_sum_body,
        out_type=jax.ShapeDtypeStruct((_SC_ROWS,), jnp.float32),
        mesh=mesh,
        compiler_params=pltpu.CompilerParams(needs_layout_passes=False),
        scratch_types=[
            pltpu.VMEM((_N,), jnp.float32),
            pltpu.VMEM((_RC, _N), jnp.int32),
            pltpu.VMEM((_RC, _N), jnp.int32),
            pltpu.VMEM((_RPW,), jnp.float32),
            pltpu.VMEM((_LANES, _LANES), jnp.float32),
            pltpu.SemaphoreType.DMA,
            pltpu.SemaphoreType.DMA,
        ],
    )(adj_1to2, f1_row)


def _tc_main_body(adj21, adj12, f2, f1, x1, x2b, w1s, w1n, w2s, w2n,
                  out1, out2b):
    a21 = adj21[...]
    vals = jnp.where(a21 != 0, f2[...], NEG)
    m = jnp.max(vals, axis=1, keepdims=True)
    h = jnp.max(a21, axis=1, keepdims=True)
    s = jnp.sum(jnp.where(adj12[...] != 0, f1[...], 0.0), axis=1, keepdims=True)

    scal1 = jnp.where(h > 0, m, 0.0)
    wsum1 = jnp.sum(w1n[...], axis=1)
    wsum2 = jnp.sum(w2n[...], axis=1)
    o1 = jnp.dot(x1[...], w1s[...].T, preferred_element_type=jnp.float32)
    o2 = jnp.dot(x2b[...], w2s[...].T, preferred_element_type=jnp.float32)
    out1[...] = jnp.maximum(o1 + scal1 * wsum1[None, :], 0.0)
    out2b[...] = jnp.maximum(o2 + s * wsum2[None, :], 0.0)


def _tc_out2_body(scal2, x2, w2s, w2n, out2):
    wsum2 = jnp.sum(w2n[...], axis=1)
    o2 = jnp.dot(x2[...], w2s[...].T, preferred_element_type=jnp.float32)
    out2[...] = jnp.maximum(o2 + scal2[...] * wsum2[None, :], 0.0)


def kernel(node_feats1, node_feats2, adj_1to2, adj_2to1,
           W1_self, W1_neigh, W2_self, W2_neigh):
    n1, d_in = node_feats1.shape
    n2, _ = node_feats2.shape
    d_out = W1_self.shape[0]

    f1_row = node_feats1[:, 0]
    f2_row = node_feats2[:, 0].reshape(1, n2)

    scal2_top = _sc_scal2_top(adj_1to2, f1_row)

    nr = n1 // _BR
    rb2 = _SC_ROWS // _BR2

    out1, out2b = pl.pallas_call(
        _tc_main_body,
        grid=(nr,),
        in_specs=[
            pl.BlockSpec((_BR, n2), lambda r: (r, 0)),
            pl.BlockSpec((_BR2, n1), lambda r: (rb2 + r, 0)),
            pl.BlockSpec((1, n2), lambda r: (0, 0)),
            pl.BlockSpec((1, n1), lambda r: (0, 0)),
            pl.BlockSpec((_BR, d_in), lambda r: (r, 0)),
            pl.BlockSpec((_BR2, d_in), lambda r: (rb2 + r, 0)),
            pl.BlockSpec((d_out, d_in), lambda r: (0, 0)),
            pl.BlockSpec((d_out, d_in), lambda r: (0, 0)),
            pl.BlockSpec((d_out, d_in), lambda r: (0, 0)),
            pl.BlockSpec((d_out, d_in), lambda r: (0, 0)),
        ],
        out_specs=[
            pl.BlockSpec((_BR, d_out), lambda r: (r, 0)),
            pl.BlockSpec((_BR2, d_out), lambda r: (r, 0)),
        ],
        out_shape=[
            jax.ShapeDtypeStruct((n1, d_out), jnp.float32),
            jax.ShapeDtypeStruct((n2 - _SC_ROWS, d_out), jnp.float32),
        ],
        compiler_params=pltpu.CompilerParams(
            dimension_semantics=("parallel",),
        ),
    )(adj_2to1, adj_1to2, f2_row, f1_row.reshape(1, n1), node_feats1,
      node_feats2, W1_self, W1_neigh, W2_self, W2_neigh)

    br2 = 512
    out2t = pl.pallas_call(
        _tc_out2_body,
        grid=(_SC_ROWS // br2,),
        in_specs=[
            pl.BlockSpec((br2, 1), lambda r: (r, 0)),
            pl.BlockSpec((br2, d_in), lambda r: (r, 0)),
            pl.BlockSpec((d_out, d_in), lambda r: (0, 0)),
            pl.BlockSpec((d_out, d_in), lambda r: (0, 0)),
        ],
        out_specs=pl.BlockSpec((br2, d_out), lambda r: (r, 0)),
        out_shape=jax.ShapeDtypeStruct((_SC_ROWS, d_out), jnp.float32),
        compiler_params=pltpu.CompilerParams(
            dimension_semantics=("arbitrary",),
        ),
    )(scal2_top.reshape(_SC_ROWS, 1), node_feats2, W2_self, W2_neigh)

    out2 = jnp.concatenate([out2t, out2b], axis=0)
    return out1, out2


# speedup vs baseline: 1.0076x; 1.0076x over previous
import functools
import jax
import jax.numpy as jnp
from jax import lax
from jax.experimental import pallas as pl
from jax.experimental.pallas import tpu as pltpu
from jax.experimental.pallas import tpu_sc as plsc

NEG = jnp.finfo(jnp.float32).min

_N = 8192
_NW = 32
_SC_ROWS = 2048
_RPW = _SC_ROWS // _NW
_RC = 4
_NCH = _RPW // _RC
_LANES = 16
_KV = _N // _LANES
_KU = 4

_BR = 256
_BR2 = (_N - _SC_ROWS) // (_N // _BR)


def _sc_sum_body(adj_hbm, f1_hbm, out_hbm, f1_v, buf0, buf1, out_v, acc_buf,
                 sem0, sem1):
    wid = lax.axis_index("s") * 2 + lax.axis_index("c")
    base = (_N - _SC_ROWS) + wid * _RPW
    pltpu.sync_copy(f1_hbm, f1_v)
    bufs = (buf0, buf1)
    sems = (sem0, sem1)
    lanes = lax.iota(jnp.int32, _LANES)

    pltpu.async_copy(adj_hbm.at[pl.ds(base, _RC)], buf0, sem0)

    def group_body(g, _):
        for cc in range(_LANES // _RC):
            c = g * (_LANES // _RC) + cc
            p = cc % 2
            buf = bufs[p]
            row0 = base + c * _RC
            pltpu.make_async_copy(adj_hbm.at[pl.ds(row0, _RC)], buf, sems[p]).wait()

            @pl.when(c + 1 < _NCH)
            def _prefetch():
                pltpu.async_copy(
                    adj_hbm.at[pl.ds(row0 + _RC, _RC)], bufs[1 - p], sems[1 - p])

            def kbody(k, accs):
                accs = list(accs)
                for u in range(_KU):
                    off = (k * _KU + u) * _LANES
                    f = f1_v[pl.ds(off, _LANES)]
                    for r in range(_RC):
                        a = buf[r, pl.ds(off, _LANES)].astype(jnp.float32)
                        accs[r] = accs[r] + a * f
                return tuple(accs)

            accs = lax.fori_loop(
                0, _KV // _KU, kbody, tuple(jnp.zeros((_LANES,), jnp.float32)
                                            for _ in range(_RC)))
            for r in range(_RC):
                acc_buf[cc * _RC + r, :] = accs[r]
        res = jnp.zeros((_LANES,), jnp.float32)
        for t in range(_LANES):
            col = jnp.full((_LANES,), t, jnp.int32)
            res = res + plsc.load_gather(acc_buf, [lanes, col])
        out_v[pl.ds(g * _LANES, _LANES)] = res
        return 0

    lax.fori_loop(0, _RPW // _LANES, group_body, 0)
    pltpu.sync_copy(out_v, out_hbm.at[pl.ds(base - (_N - _SC_ROWS), _RPW)])


def _sc_scal2_top(adj_1to2, f1_row):
    mesh = plsc.VectorSubcoreMesh(core_axis_name="c", subcore_axis_name="s")
    return pl.kernel(
        _sc_sum_body,
        out_type=jax.ShapeDtypeStruct((_SC_ROWS,), jnp.float32),
        mesh=mesh,
        compiler_params=pltpu.CompilerParams(needs_layout_passes=False),
        scratch_types=[
            pltpu.VMEM((_N,), jnp.float32),
            pltpu.VMEM((_RC, _N), jnp.int32),
            pltpu.VMEM((_RC, _N), jnp.int32),
            pltpu.VMEM((_RPW,), jnp.float32),
            pltpu.VMEM((_LANES, _LANES), jnp.float32),
            pltpu.SemaphoreType.DMA,
            pltpu.SemaphoreType.DMA,
        ],
    )(adj_1to2, f1_row)


def _tc_main_body(adj21, adj12, f2, f1, x1, x2b, w1s, w1n, w2s, w2n,
                  out1, out2b):
    a21 = adj21[...]
    vals = jnp.where(a21 != 0, f2[...], NEG)
    m = jnp.max(vals, axis=1, keepdims=True)
    h = jnp.max(a21, axis=1, keepdims=True)
    s = jnp.sum(jnp.where(adj12[...] != 0, f1[...], 0.0), axis=1, keepdims=True)

    scal1 = jnp.where(h > 0, m, 0.0)
    wsum1 = jnp.sum(w1n[...], axis=1)
    wsum2 = jnp.sum(w2n[...], axis=1)
    o1 = jnp.dot(x1[...], w1s[...].T, preferred_element_type=jnp.float32)
    o2 = jnp.dot(x2b[...], w2s[...].T, preferred_element_type=jnp.float32)
    out1[...] = jnp.maximum(o1 + scal1 * wsum1[None, :], 0.0)
    out2b[...] = jnp.maximum(o2 + s * wsum2[None, :], 0.0)


def _tc_out2_body(scal2, x2, w2s, w2n, out2):
    wsum2 = jnp.sum(w2n[...], axis=1)
    o2 = jnp.dot(x2[...], w2s[...].T, preferred_element_type=jnp.float32)
    out2[...] = jnp.maximum(o2 + scal2[...] * wsum2[None, :], 0.0)


def kernel(node_feats1, node_feats2, adj_1to2, adj_2to1,
           W1_self, W1_neigh, W2_self, W2_neigh):
    n1, d_in = node_feats1.shape
    n2, _ = node_feats2.shape
    d_out = W1_self.shape[0]

    f1_row = node_feats1[:, 0]
    f2_row = node_feats2[:, 0].reshape(1, n2)

    scal2_top = _sc_scal2_top(adj_1to2, f1_row)

    nr = n1 // _BR

    out1, out2b = pl.pallas_call(
        _tc_main_body,
        grid=(nr,),
        in_specs=[
            pl.BlockSpec((_BR, n2), lambda r: (r, 0)),
            pl.BlockSpec((_BR2, n1), lambda r: (r, 0)),
            pl.BlockSpec((1, n2), lambda r: (0, 0)),
            pl.BlockSpec((1, n1), lambda r: (0, 0)),
            pl.BlockSpec((_BR, d_in), lambda r: (r, 0)),
            pl.BlockSpec((_BR2, d_in), lambda r: (r, 0)),
            pl.BlockSpec((d_out, d_in), lambda r: (0, 0)),
            pl.BlockSpec((d_out, d_in), lambda r: (0, 0)),
            pl.BlockSpec((d_out, d_in), lambda r: (0, 0)),
            pl.BlockSpec((d_out, d_in), lambda r: (0, 0)),
        ],
        out_specs=[
            pl.BlockSpec((_BR, d_out), lambda r: (r, 0)),
            pl.BlockSpec((_BR2, d_out), lambda r: (r, 0)),
        ],
        out_shape=[
            jax.ShapeDtypeStruct((n1, d_out), jnp.float32),
            jax.ShapeDtypeStruct((n2 - _SC_ROWS, d_out), jnp.float32),
        ],
        compiler_params=pltpu.CompilerParams(
            dimension_semantics=("parallel",),
        ),
    )(adj_2to1, adj_1to2, f2_row, f1_row.reshape(1, n1), node_feats1,
      node_feats2, W1_self, W1_neigh, W2_self, W2_neigh)

    br2 = 512
    out2t = pl.pallas_call(
        _tc_out2_body,
        grid=(_SC_ROWS // br2,),
        in_specs=[
            pl.BlockSpec((br2, 1), lambda r: (r, 0)),
            pl.BlockSpec((br2, d_in), lambda r: (r + (_N - _SC_ROWS) // 512, 0)),
            pl.BlockSpec((d_out, d_in), lambda r: (0, 0)),
            pl.BlockSpec((d_out, d_in), lambda r: (0, 0)),
        ],
        out_specs=pl.BlockSpec((br2, d_out), lambda r: (r, 0)),
        out_shape=jax.ShapeDtypeStruct((_SC_ROWS, d_out), jnp.float32),
        compiler_params=pltpu.CompilerParams(
            dimension_semantics=("arbitrary",),
        ),
    )(scal2_top.reshape(_SC_ROWS, 1), node_feats2, W2_self, W2_neigh)

    out2 = jnp.concatenate([out2b, out2t], axis=0)
    return out1, out2


# speedup vs baseline: 1.1407x; 1.1322x over previous
import jax
import jax.numpy as jnp
from jax.experimental import pallas as pl
from jax.experimental.pallas import tpu as pltpu

NEG = jnp.finfo(jnp.float32).min


def _body(adj21, adj12, f2, f1, x1, x2, w1s, w1n, w2s, w2n, out1, out2):
    a21 = adj21[...]
    a12 = adj12[...]
    vals = jnp.where(a21 != 0, f2[...], NEG)
    m = jnp.max(vals, axis=1, keepdims=True)
    h = jnp.max(a21, axis=1, keepdims=True)
    s = jnp.sum(jnp.where(a12 != 0, f1[...], 0.0), axis=1, keepdims=True)

    scal1 = jnp.where(h > 0, m, 0.0)
    wsum1 = jnp.sum(w1n[...], axis=1)
    wsum2 = jnp.sum(w2n[...], axis=1)
    o1 = jnp.dot(x1[...], w1s[...].T, preferred_element_type=jnp.float32)
    o2 = jnp.dot(x2[...], w2s[...].T, preferred_element_type=jnp.float32)
    out1[...] = jnp.maximum(o1 + scal1 * wsum1[None, :], 0.0)
    out2[...] = jnp.maximum(o2 + s * wsum2[None, :], 0.0)


def kernel(node_feats1, node_feats2, adj_1to2, adj_2to1,
           W1_self, W1_neigh, W2_self, W2_neigh):
    n1, d_in = node_feats1.shape
    n2, _ = node_feats2.shape
    d_out = W1_self.shape[0]

    br = 128
    nr = n1 // br

    f2_row = node_feats2[:, 0].reshape(1, n2)
    f1_row = node_feats1[:, 0].reshape(1, n1)

    out1, out2 = pl.pallas_call(
        _body,
        grid=(nr,),
        in_specs=[
            pl.BlockSpec((br, n2), lambda r: (r, 0)),
            pl.BlockSpec((br, n1), lambda r: (r, 0)),
            pl.BlockSpec((1, n2), lambda r: (0, 0)),
            pl.BlockSpec((1, n1), lambda r: (0, 0)),
            pl.BlockSpec((br, d_in), lambda r: (r, 0)),
            pl.BlockSpec((br, d_in), lambda r: (r, 0)),
            pl.BlockSpec((d_out, d_in), lambda r: (0, 0)),
            pl.BlockSpec((d_out, d_in), lambda r: (0, 0)),
            pl.BlockSpec((d_out, d_in), lambda r: (0, 0)),
            pl.BlockSpec((d_out, d_in), lambda r: (0, 0)),
        ],
        out_specs=[
            pl.BlockSpec((br, d_out), lambda r: (r, 0)),
            pl.BlockSpec((br, d_out), lambda r: (r, 0)),
        ],
        out_shape=[
            jax.ShapeDtypeStruct((n1, d_out), jnp.float32),
            jax.ShapeDtypeStruct((n2, d_out), jnp.float32),
        ],
        compiler_params=pltpu.CompilerParams(
            dimension_semantics=("parallel",),
        ),
    )(adj_2to1, adj_1to2, f2_row, f1_row, node_feats1, node_feats2,
      W1_self, W1_neigh, W2_self, W2_neigh)
    return out1, out2


# speedup vs baseline: 1.1420x; 1.0011x over previous
import jax
import jax.numpy as jnp
from jax.experimental import pallas as pl
from jax.experimental.pallas import tpu as pltpu

NEG = jnp.finfo(jnp.float32).min


def _body(adj21, adj12, f2, f1, x1, x2, w1s, w1n, w2s, w2n, out1, out2):
    a21 = adj21[...]
    a12 = adj12[...]
    vals = jnp.where(a21 != 0, f2[...], NEG)
    m = jnp.max(vals, axis=1, keepdims=True)
    h = jnp.max(a21, axis=1, keepdims=True)
    s = jnp.sum(jnp.where(a12 != 0, f1[...], 0.0), axis=1, keepdims=True)

    scal1 = jnp.where(h > 0, m, 0.0)
    wsum1 = jnp.sum(w1n[...], axis=1)
    wsum2 = jnp.sum(w2n[...], axis=1)
    o1 = jnp.dot(x1[...], w1s[...].T, preferred_element_type=jnp.float32)
    o2 = jnp.dot(x2[...], w2s[...].T, preferred_element_type=jnp.float32)
    out1[...] = jnp.maximum(o1 + scal1 * wsum1[None, :], 0.0)
    out2[...] = jnp.maximum(o2 + s * wsum2[None, :], 0.0)


def kernel(node_feats1, node_feats2, adj_1to2, adj_2to1,
           W1_self, W1_neigh, W2_self, W2_neigh):
    n1, d_in = node_feats1.shape
    n2, _ = node_feats2.shape
    d_out = W1_self.shape[0]

    br = 256
    nr = n1 // br

    f2_row = node_feats2[:, 0].reshape(1, n2)
    f1_row = node_feats1[:, 0].reshape(1, n1)

    out1, out2 = pl.pallas_call(
        _body,
        grid=(nr,),
        in_specs=[
            pl.BlockSpec((br, n2), lambda r: (r, 0)),
            pl.BlockSpec((br, n1), lambda r: (r, 0)),
            pl.BlockSpec((1, n2), lambda r: (0, 0)),
            pl.BlockSpec((1, n1), lambda r: (0, 0)),
            pl.BlockSpec((br, d_in), lambda r: (r, 0)),
            pl.BlockSpec((br, d_in), lambda r: (r, 0)),
            pl.BlockSpec((d_out, d_in), lambda r: (0, 0)),
            pl.BlockSpec((d_out, d_in), lambda r: (0, 0)),
            pl.BlockSpec((d_out, d_in), lambda r: (0, 0)),
            pl.BlockSpec((d_out, d_in), lambda r: (0, 0)),
        ],
        out_specs=[
            pl.BlockSpec((br, d_out), lambda r: (r, 0)),
            pl.BlockSpec((br, d_out), lambda r: (r, 0)),
        ],
        out_shape=[
            jax.ShapeDtypeStruct((n1, d_out), jnp.float32),
            jax.ShapeDtypeStruct((n2, d_out), jnp.float32),
        ],
        compiler_params=pltpu.CompilerParams(
            dimension_semantics=("parallel",),
        ),
    )(adj_2to1, adj_1to2, f2_row, f1_row, node_feats1, node_feats2,
      W1_self, W1_neigh, W2_self, W2_neigh)
    return out1, out2


# speedup vs baseline: 1.1536x; 1.0102x over previous
import jax
import jax.numpy as jnp
from jax.experimental import pallas as pl
from jax.experimental.pallas import tpu as pltpu

NEG = jnp.finfo(jnp.float32).min


def _body(a21l, a21r, a12l, a12r, f2, f1, x1, x2, w1s, w1n, w2s, w2n,
          out1, out2):
    n = f2.shape[1]
    f2l = f2[:, : n // 2]
    f2r = f2[:, n // 2:]
    f1l = f1[:, : n // 2]
    f1r = f1[:, n // 2:]

    al = a21l[...]
    ar = a21r[...]
    ml = jnp.max(jnp.where(al != 0, f2l, NEG), axis=1, keepdims=True)
    mr = jnp.max(jnp.where(ar != 0, f2r, NEG), axis=1, keepdims=True)
    m = jnp.maximum(ml, mr)
    h = jnp.maximum(jnp.max(al, axis=1, keepdims=True),
                    jnp.max(ar, axis=1, keepdims=True))
    sl = jnp.sum(jnp.where(a12l[...] != 0, f1l, 0.0), axis=1, keepdims=True)
    sr = jnp.sum(jnp.where(a12r[...] != 0, f1r, 0.0), axis=1, keepdims=True)
    s = sl + sr

    scal1 = jnp.where(h > 0, m, 0.0)
    wsum1 = jnp.sum(w1n[...], axis=1)
    wsum2 = jnp.sum(w2n[...], axis=1)
    o1 = jnp.dot(x1[...], w1s[...].T, preferred_element_type=jnp.float32)
    o2 = jnp.dot(x2[...], w2s[...].T, preferred_element_type=jnp.float32)
    out1[...] = jnp.maximum(o1 + scal1 * wsum1[None, :], 0.0)
    out2[...] = jnp.maximum(o2 + s * wsum2[None, :], 0.0)


def kernel(node_feats1, node_feats2, adj_1to2, adj_2to1,
           W1_self, W1_neigh, W2_self, W2_neigh):
    n1, d_in = node_feats1.shape
    n2, _ = node_feats2.shape
    d_out = W1_self.shape[0]

    br = 256
    hc = n2 // 2
    nr = n1 // br

    f2_row = node_feats2[:, 0].reshape(1, n2)
    f1_row = node_feats1[:, 0].reshape(1, n1)

    out1, out2 = pl.pallas_call(
        _body,
        grid=(nr,),
        in_specs=[
            pl.BlockSpec((br, hc), lambda r: (r, 0)),
            pl.BlockSpec((br, hc), lambda r: (r, 1)),
            pl.BlockSpec((br, hc), lambda r: (r, 0)),
            pl.BlockSpec((br, hc), lambda r: (r, 1)),
            pl.BlockSpec((1, n2), lambda r: (0, 0)),
            pl.BlockSpec((1, n1), lambda r: (0, 0)),
            pl.BlockSpec((br, d_in), lambda r: (r, 0)),
            pl.BlockSpec((br, d_in), lambda r: (r, 0)),
            pl.BlockSpec((d_out, d_in), lambda r: (0, 0)),
            pl.BlockSpec((d_out, d_in), lambda r: (0, 0)),
            pl.BlockSpec((d_out, d_in), lambda r: (0, 0)),
            pl.BlockSpec((d_out, d_in), lambda r: (0, 0)),
        ],
        out_specs=[
            pl.BlockSpec((br, d_out), lambda r: (r, 0)),
            pl.BlockSpec((br, d_out), lambda r: (r, 0)),
        ],
        out_shape=[
            jax.ShapeDtypeStruct((n1, d_out), jnp.float32),
            jax.ShapeDtypeStruct((n2, d_out), jnp.float32),
        ],
        compiler_params=pltpu.CompilerParams(
            dimension_semantics=("parallel",),
        ),
    )(adj_2to1, adj_2to1, adj_1to2, adj_1to2, f2_row, f1_row,
      node_feats1, node_feats2, W1_self, W1_neigh, W2_self, W2_neigh)
    return out1, out2
